# baseline (speedup 1.0000x reference)
import functools

import jax
import jax.numpy as jnp
from jax.experimental import pallas as pl
from jax.experimental.pallas import tpu as pltpu

TOP_K = 2
TM = 256
DF_BLK = 128


def _moe_kernel(row_token_ref, tile_expert_ref, tile_flag_ref,
                x_ref, w1g_ref, w1u_ref, w2_ref, w_ref,
                out_ref, xs_ref, acc_ref, *, n_j):
    i = pl.program_id(0)
    j = pl.program_id(1)

    @pl.when(jnp.logical_and(i == 0, j == 0))
    def _zero_out():
        out_ref[...] = jnp.zeros_like(out_ref)

    @pl.when(tile_flag_ref[i] != 0)
    def _active():
        @pl.when(j == 0)
        def _gather():
            def body(r, _):
                tok = row_token_ref[i * TM + r]
                xs_ref[pl.ds(r, 1), :] = x_ref[pl.ds(tok, 1), :]
                return 0
            jax.lax.fori_loop(0, TM, body, 0, unroll=8)

        xs = xs_ref[...]
        gate = jax.lax.dot_general(
            xs, w1g_ref[0, 0], (((1,), (1,)), ((), ())),
            preferred_element_type=jnp.float32)
        up = jax.lax.dot_general(
            xs, w1u_ref[0, 0], (((1,), (1,)), ((), ())),
            preferred_element_type=jnp.float32)
        act = gate * jax.nn.sigmoid(gate) * up
        yj = jax.lax.dot_general(
            act, w2_ref[0], (((1,), (1,)), ((), ())),
            preferred_element_type=jnp.float32)

        @pl.when(j == 0)
        def _init():
            acc_ref[...] = yj

        @pl.when(j > 0)
        def _acc():
            acc_ref[...] += yj

        @pl.when(j == n_j - 1)
        def _scatter():
            acc_ref[...] *= w_ref[...]

            def body(r, _):
                tok = row_token_ref[i * TM + r]
                out_ref[pl.ds(tok, 1), :] += acc_ref[pl.ds(r, 1), :]
                return 0
            jax.lax.fori_loop(0, TM, body, 0, unroll=8)


def kernel(x, router_logits, w1, w2):
    n_tok, hidden = x.shape
    n_exp = w1.shape[0]
    d_ff = w2.shape[2]

    probs = jax.nn.softmax(router_logits.astype(jnp.float32), axis=-1)
    topk_w, topk_idx = jax.lax.top_k(probs, TOP_K)
    topk_w = topk_w / jnp.sum(topk_w, axis=-1, keepdims=True)

    n_asn = n_tok * TOP_K
    e_flat = topk_idx.reshape(-1).astype(jnp.int32)
    w_flat = topk_w.reshape(-1)
    t_flat = jnp.repeat(jnp.arange(n_tok, dtype=jnp.int32), TOP_K)

    order = jnp.argsort(e_flat)
    e_s = e_flat[order]
    t_s = t_flat[order]
    w_s = w_flat[order]

    counts = jnp.bincount(e_flat, length=n_exp)
    padded = ((counts + TM - 1) // TM) * TM
    pstart = jnp.cumsum(padded) - padded
    gstart = jnp.cumsum(counts) - counts
    rank = jnp.arange(n_asn, dtype=jnp.int32) - gstart[e_s].astype(jnp.int32)
    dest = (pstart[e_s].astype(jnp.int32) + rank)

    n_rows = n_asn + n_exp * TM
    n_tiles = n_rows // TM
    row_token = jnp.zeros((n_rows,), jnp.int32).at[dest].set(t_s)
    row_weight = jnp.zeros((n_rows, 1), jnp.float32).at[dest, 0].set(w_s)

    tile_start = jnp.arange(n_tiles, dtype=jnp.int32) * TM
    total_padded = jnp.sum(padded).astype(jnp.int32)
    tile_flag = (tile_start < total_padded).astype(jnp.int32)
    pend = (pstart + padded).astype(jnp.int32)
    tile_expert = jnp.clip(
        jnp.searchsorted(pend, tile_start, side='right'), 0, n_exp - 1
    ).astype(jnp.int32)

    w1r = w1.reshape(n_exp, 2, d_ff, hidden)
    n_j = d_ff // DF_BLK

    grid_spec = pltpu.PrefetchScalarGridSpec(
        num_scalar_prefetch=3,
        grid=(n_tiles, n_j),
        in_specs=[
            pl.BlockSpec((n_tok, hidden), lambda i, j, rt, te, tf: (0, 0)),
            pl.BlockSpec((1, 1, DF_BLK, hidden),
                         lambda i, j, rt, te, tf: (te[i], 0, j, 0)),
            pl.BlockSpec((1, 1, DF_BLK, hidden),
                         lambda i, j, rt, te, tf: (te[i], 1, j, 0)),
            pl.BlockSpec((1, hidden, DF_BLK),
                         lambda i, j, rt, te, tf: (te[i], 0, j)),
            pl.BlockSpec((TM, 1), lambda i, j, rt, te, tf: (i, 0)),
        ],
        out_specs=pl.BlockSpec((n_tok, hidden), lambda i, j, rt, te, tf: (0, 0)),
        scratch_shapes=[
            pltpu.VMEM((TM, hidden), jnp.float32),
            pltpu.VMEM((TM, hidden), jnp.float32),
        ],
    )

    out = pl.pallas_call(
        functools.partial(_moe_kernel, n_j=n_j),
        grid_spec=grid_spec,
        out_shape=jax.ShapeDtypeStruct((n_tok, hidden), jnp.float32),
        compiler_params=pltpu.CompilerParams(
            dimension_semantics=("arbitrary", "arbitrary")),
    )(row_token, tile_expert, tile_flag, x, w1r, w1r, w2, row_weight)
    return out
